# Initial kernel scaffold; baseline (speedup 1.0000x reference)
import jax
import jax.numpy as jnp
from jax.experimental import pallas as pl


def kernel(inputs, emb_table, W_h, b_h, W_o, b_o):
    raise NotImplementedError("write your pallas kernel here")


# speedup vs baseline: 11.9393x; 11.9393x over previous
import functools

import jax
import jax.numpy as jnp
from jax import lax
from jax.experimental import pallas as pl
from jax.experimental.pallas import tpu as pltpu
from jax.experimental.pallas import tpu_sc as plsc

EMB = 128
HID = 128
OUT = 91


def _sc_gather(table, idx_flat):
    n = idx_flat.shape[0]
    info = plsc.get_sparse_core_info()
    nw = info.num_cores * info.num_subcores
    b_per_w = n // nw
    assert b_per_w * nw == n
    ch = 384
    n_ch = b_per_w // ch
    assert n_ch * ch == b_per_w

    mesh = plsc.VectorSubcoreMesh(core_axis_name="c", subcore_axis_name="s")

    @functools.partial(
        pl.kernel,
        mesh=mesh,
        out_type=jax.ShapeDtypeStruct((n, EMB), jnp.float32),
        scratch_types=[
            pltpu.VMEM((b_per_w,), jnp.int32),
            pltpu.VMEM((2, ch, EMB), jnp.float32),
            pltpu.SemaphoreType.DMA,
            pltpu.SemaphoreType.DMA,
            pltpu.SemaphoreType.DMA,
            pltpu.SemaphoreType.DMA,
        ],
    )
    def k(table_hbm, idx_hbm, out_hbm, idx_v, rows_v, g0, g1, w0, w1):
        wid = lax.axis_index("s") * info.num_cores + lax.axis_index("c")
        base = wid * b_per_w
        pltpu.sync_copy(idx_hbm.at[pl.ds(base, b_per_w)], idx_v)
        gsem = (g0, g1)
        wsem = (w0, w1)
        gcp = [None] * n_ch
        wcp = [None] * n_ch

        def start_gather(c):
            gcp[c] = pltpu.async_copy(
                table_hbm.at[idx_v.at[pl.ds(c * ch, ch)]],
                rows_v.at[c % 2],
                gsem[c % 2],
            )

        start_gather(0)
        if n_ch > 1:
            start_gather(1)
        for c in range(n_ch):
            gcp[c].wait()
            wcp[c] = pltpu.async_copy(
                rows_v.at[c % 2],
                out_hbm.at[pl.ds(base + c * ch, ch)],
                wsem[c % 2],
            )
            if c + 2 < n_ch:
                wcp[c].wait()
                start_gather(c + 2)
        for c in range(max(0, n_ch - 2), n_ch):
            wcp[c].wait()

    return k(table, idx_flat)


def _mlp_body(x_ref, wh_ref, bh_ref, wo_ref, bo_ref, out_ref):
    h = jnp.dot(x_ref[...], wh_ref[...], preferred_element_type=jnp.float32)
    h = jnp.maximum(h + bh_ref[...], 0.0)
    out_ref[...] = (
        jnp.dot(h, wo_ref[...], preferred_element_type=jnp.float32) + bo_ref[...]
    )


def _tc_mlp(x, W_h, b_h, W_o, b_o):
    bq, d = x.shape
    blk = 1024
    grid = bq // blk
    return pl.pallas_call(
        _mlp_body,
        grid=(grid,),
        in_specs=[
            pl.BlockSpec((blk, d), lambda i: (i, 0)),
            pl.BlockSpec((d, HID), lambda i: (0, 0)),
            pl.BlockSpec((1, HID), lambda i: (0, 0)),
            pl.BlockSpec((HID, OUT), lambda i: (0, 0)),
            pl.BlockSpec((1, OUT), lambda i: (0, 0)),
        ],
        out_specs=pl.BlockSpec((blk, OUT), lambda i: (i, 0)),
        out_shape=jax.ShapeDtypeStruct((bq, OUT), jnp.float32),
    )(x, W_h, b_h.reshape(1, HID), W_o, b_o.reshape(1, OUT))


def kernel(inputs, emb_table, W_h, b_h, W_o, b_o):
    bq, ctx = inputs.shape
    idx_flat = inputs.reshape(-1)
    rows = _sc_gather(emb_table, idx_flat)
    x = rows.reshape(bq, ctx * EMB)
    return _tc_mlp(x, W_h, b_h, W_o, b_o)


# speedup vs baseline: 18.3324x; 1.5355x over previous
import functools

import jax
import jax.numpy as jnp
from jax import lax
from jax.experimental import pallas as pl
from jax.experimental.pallas import tpu as pltpu
from jax.experimental.pallas import tpu_sc as plsc

EMB = 128
HID = 128
OUT = 91


def _sc_gather(table, idx_flat):
    n = idx_flat.shape[0]
    info = plsc.get_sparse_core_info()
    nw = info.num_cores * info.num_subcores
    b_per_w = n // nw
    assert b_per_w * nw == n
    ch = 384
    n_ch = b_per_w // ch
    assert n_ch * ch == b_per_w

    mesh = plsc.VectorSubcoreMesh(core_axis_name="c", subcore_axis_name="s")

    @functools.partial(
        pl.kernel,
        mesh=mesh,
        out_type=jax.ShapeDtypeStruct((n, EMB), jnp.float32),
        scratch_types=[
            pltpu.VMEM((b_per_w,), jnp.int32),
            pltpu.VMEM((2, ch, EMB), jnp.float32),
            pltpu.SemaphoreType.DMA,
            pltpu.SemaphoreType.DMA,
            pltpu.SemaphoreType.DMA,
            pltpu.SemaphoreType.DMA,
        ],
    )
    def k(table_hbm, idx_hbm, out_hbm, idx_v, rows_v, g0, g1, w0, w1):
        wid = lax.axis_index("s") * info.num_cores + lax.axis_index("c")
        base = wid * b_per_w
        pltpu.sync_copy(idx_hbm.at[pl.ds(base, b_per_w)], idx_v)
        gsem = (g0, g1)
        wsem = (w0, w1)
        gcp = [None] * n_ch
        wcp = [None] * n_ch

        def start_gather(c):
            gcp[c] = pltpu.async_copy(
                table_hbm.at[idx_v.at[pl.ds(c * ch, ch)]],
                rows_v.at[c % 2],
                gsem[c % 2],
            )

        start_gather(0)
        if n_ch > 1:
            start_gather(1)
        for c in range(n_ch):
            gcp[c].wait()
            wcp[c] = pltpu.async_copy(
                rows_v.at[c % 2],
                out_hbm.at[pl.ds(base + c * ch, ch)],
                wsem[c % 2],
            )
            if c + 2 < n_ch:
                wcp[c].wait()
                start_gather(c + 2)
        for c in range(max(0, n_ch - 2), n_ch):
            wcp[c].wait()

    return k(table, idx_flat)


def _mlp_body(x_ref, w3_ref, bh_ref, wo_ref, bo_ref, out_ref):
    ctx = x_ref.shape[0]
    acc = jnp.dot(x_ref[0], w3_ref[0], preferred_element_type=jnp.float32)
    for c in range(1, ctx):
        acc += jnp.dot(x_ref[c], w3_ref[c], preferred_element_type=jnp.float32)
    h = jnp.maximum(acc + bh_ref[...], 0.0)
    out_ref[...] = (
        jnp.dot(h, wo_ref[...], preferred_element_type=jnp.float32) + bo_ref[...]
    )


def _tc_mlp(x, W_h, b_h, W_o, b_o):
    ctx, bq, _ = x.shape
    blk = 512
    grid = bq // blk
    return pl.pallas_call(
        _mlp_body,
        grid=(grid,),
        in_specs=[
            pl.BlockSpec((ctx, blk, EMB), lambda i: (0, i, 0)),
            pl.BlockSpec((ctx, EMB, HID), lambda i: (0, 0, 0)),
            pl.BlockSpec((1, HID), lambda i: (0, 0)),
            pl.BlockSpec((HID, OUT), lambda i: (0, 0)),
            pl.BlockSpec((1, OUT), lambda i: (0, 0)),
        ],
        out_specs=pl.BlockSpec((blk, OUT), lambda i: (i, 0)),
        out_shape=jax.ShapeDtypeStruct((bq, OUT), jnp.float32),
    )(x, W_h, b_h.reshape(1, HID), W_o, b_o.reshape(1, OUT))


def kernel(inputs, emb_table, W_h, b_h, W_o, b_o):
    bq, ctx = inputs.shape
    idx_flat = inputs.T.reshape(-1)
    rows = _sc_gather(emb_table, idx_flat)
    x = rows.reshape(ctx, bq, EMB)
    w3 = W_h.reshape(ctx, EMB, HID)
    return _tc_mlp(x, w3, b_h, W_o, b_o)


# speedup vs baseline: 20.6318x; 1.1254x over previous
import functools

import jax
import jax.numpy as jnp
from jax import lax
from jax.experimental import pallas as pl
from jax.experimental.pallas import tpu as pltpu
from jax.experimental.pallas import tpu_sc as plsc

EMB = 128
HID = 128
OUT = 91


def _sc_gather(table, idx_flat):
    n = idx_flat.shape[0]
    info = plsc.get_sparse_core_info()
    nw = info.num_cores * info.num_subcores
    b_per_w = n // nw
    assert b_per_w * nw == n
    ch = 256
    nbuf = 3
    n_ch = b_per_w // ch
    assert n_ch * ch == b_per_w

    mesh = plsc.VectorSubcoreMesh(core_axis_name="c", subcore_axis_name="s")

    @functools.partial(
        pl.kernel,
        mesh=mesh,
        out_type=jax.ShapeDtypeStruct((n, EMB), jnp.float32),
        scratch_types=[
            pltpu.VMEM((b_per_w,), jnp.int32),
            pltpu.VMEM((nbuf, ch, EMB), jnp.float32),
            pltpu.SemaphoreType.DMA,
            pltpu.SemaphoreType.DMA,
            pltpu.SemaphoreType.DMA,
            pltpu.SemaphoreType.DMA,
            pltpu.SemaphoreType.DMA,
            pltpu.SemaphoreType.DMA,
        ],
    )
    def k(table_hbm, idx_hbm, out_hbm, idx_v, rows_v, g0, g1, g2, w0, w1, w2):
        wid = lax.axis_index("s") * info.num_cores + lax.axis_index("c")
        base = wid * b_per_w
        pltpu.sync_copy(idx_hbm.at[pl.ds(base, b_per_w)], idx_v)
        gsem = (g0, g1, g2)
        wsem = (w0, w1, w2)
        gcp = [None] * n_ch
        wcp = [None] * n_ch

        def start_gather(c):
            gcp[c] = pltpu.async_copy(
                table_hbm.at[idx_v.at[pl.ds(c * ch, ch)]],
                rows_v.at[c % nbuf],
                gsem[c % nbuf],
            )

        for c in range(min(nbuf, n_ch)):
            start_gather(c)
        for c in range(n_ch):
            gcp[c].wait()
            wcp[c] = pltpu.async_copy(
                rows_v.at[c % nbuf],
                out_hbm.at[pl.ds(base + c * ch, ch)],
                wsem[c % nbuf],
            )
            if c + nbuf < n_ch:
                wcp[c].wait()
                start_gather(c + nbuf)
        for c in range(max(0, n_ch - nbuf), n_ch):
            wcp[c].wait()

    return k(table, idx_flat)


def _mlp_body(x_ref, w3_ref, bh_ref, wo_ref, bo_ref, out_ref):
    ctx = x_ref.shape[0]
    acc = jnp.dot(x_ref[0], w3_ref[0], preferred_element_type=jnp.float32)
    for c in range(1, ctx):
        acc += jnp.dot(x_ref[c], w3_ref[c], preferred_element_type=jnp.float32)
    h = jnp.maximum(acc + bh_ref[...], 0.0)
    out_ref[...] = (
        jnp.dot(h, wo_ref[...], preferred_element_type=jnp.float32) + bo_ref[...]
    )


def _tc_mlp(x, W_h, b_h, W_o, b_o):
    ctx, bq, _ = x.shape
    blk = 1024
    grid = bq // blk
    return pl.pallas_call(
        _mlp_body,
        grid=(grid,),
        in_specs=[
            pl.BlockSpec((ctx, blk, EMB), lambda i: (0, i, 0)),
            pl.BlockSpec((ctx, EMB, HID), lambda i: (0, 0, 0)),
            pl.BlockSpec((1, HID), lambda i: (0, 0)),
            pl.BlockSpec((HID, OUT), lambda i: (0, 0)),
            pl.BlockSpec((1, OUT), lambda i: (0, 0)),
        ],
        out_specs=pl.BlockSpec((blk, OUT), lambda i: (i, 0)),
        out_shape=jax.ShapeDtypeStruct((bq, OUT), jnp.float32),
    )(x, W_h, b_h.reshape(1, HID), W_o, b_o.reshape(1, OUT))


def kernel(inputs, emb_table, W_h, b_h, W_o, b_o):
    bq, ctx = inputs.shape
    idx_flat = inputs.T.reshape(-1)
    rows = _sc_gather(emb_table, idx_flat)
    x = rows.reshape(ctx, bq, EMB)
    w3 = W_h.reshape(ctx, EMB, HID)
    return _tc_mlp(x, w3, b_h, W_o, b_o)
